# Initial kernel scaffold; baseline (speedup 1.0000x reference)
import jax
import jax.numpy as jnp
from jax.experimental import pallas as pl


def kernel(feature, edge_index, W1, b1, W2, b2):
    raise NotImplementedError("write your pallas kernel here")


# speedup vs baseline: 20.0884x; 20.0884x over previous
import functools

import jax
import jax.numpy as jnp
from jax import lax
from jax.experimental import pallas as pl
from jax.experimental.pallas import tpu as pltpu
from jax.experimental.pallas import tpu_sc as plsc

N = 10000
E = 320000
D = 128

NC = 2
NS = 16
NW = NC * NS
L = 16

BLK = 128
BPW = 80
NBLKS = BPW * NW
EPAD = NBLKS * BLK

NPAD = 10240
SENT = NPAD - 1
SLC = NPAD // NS

@functools.cache
def _mesh():
    return plsc.VectorSubcoreMesh(
        core_axis_name="c", subcore_axis_name="s", num_cores=NC, num_subcores=NS
    )


def _fill_f32(ref, nrows, value):
    vec = jnp.full((L,), value, jnp.float32)

    def body(j, _):
        for i in range(BLK // L):
            ref[j, pl.ds(i * L, L)] = vec
        return 0

    lax.fori_loop(0, nrows, body, 0)


def _fill_1d_f32(ref, n, value):
    vec = jnp.full((L,), value, jnp.float32)

    def body(i, _):
        ref[pl.ds(i * L, L)] = vec
        return 0

    lax.fori_loop(0, n // L, body, 0)


def _hist_body(src_hbm, dst_hbm, out_hbm, idx_s, idx_d, ones, zbuf, sh_s, sh_d):
    cid = lax.axis_index("c")
    sid = lax.axis_index("s")
    wid = cid * NS + sid

    _fill_f32(ones, BPW, 1.0)
    _fill_1d_f32(zbuf, SLC, 0.0)

    pltpu.sync_copy(zbuf, sh_s.at[pl.ds(sid * SLC, SLC)])
    pltpu.sync_copy(zbuf, sh_d.at[pl.ds(sid * SLC, SLC)])

    start = wid * BPW
    pltpu.sync_copy(src_hbm.at[pl.ds(start, BPW)], idx_s)
    pltpu.sync_copy(dst_hbm.at[pl.ds(start, BPW)], idx_d)

    plsc.subcore_barrier()

    def scat(j, _):
        pltpu.sync_copy(ones.at[j], sh_s.at[idx_s.at[j]], add=True)
        pltpu.sync_copy(ones.at[j], sh_d.at[idx_d.at[j]], add=True)
        return 0

    lax.fori_loop(0, BPW, scat, 0)

    plsc.subcore_barrier()

    pltpu.sync_copy(sh_s.at[pl.ds(sid * SLC, SLC)],
                    out_hbm.at[cid, 0, pl.ds(sid * SLC, SLC)])
    pltpu.sync_copy(sh_d.at[pl.ds(sid * SLC, SLC)],
                    out_hbm.at[cid, 1, pl.ds(sid * SLC, SLC)])


@functools.cache
def _hist_kernel():
    return pl.kernel(
        _hist_body,
        out_type=jax.ShapeDtypeStruct((NC, 2, NPAD), jnp.float32),
        mesh=_mesh(),
        scratch_types=[
            pltpu.VMEM((BPW, BLK), jnp.int32),
            pltpu.VMEM((BPW, BLK), jnp.int32),
            pltpu.VMEM((BPW, BLK), jnp.float32),
            pltpu.VMEM((SLC,), jnp.float32),
            pltpu.VMEM_SHARED((NPAD,), jnp.float32),
            pltpu.VMEM_SHARED((NPAD,), jnp.float32),
        ],
    )


def _cscatter_body(src_hbm, dst_hbm, nin_hbm, out_hbm,
                   idx_s, idx_d, vals, stage, sh_nin, sh_c):
    cid = lax.axis_index("c")
    sid = lax.axis_index("s")
    wid = cid * NS + sid

    _fill_1d_f32(stage, SLC, 0.0)
    pltpu.sync_copy(stage, sh_c.at[pl.ds(sid * SLC, SLC)])
    pltpu.sync_copy(nin_hbm.at[pl.ds(sid * SLC, SLC)], stage)
    pltpu.sync_copy(stage, sh_nin.at[pl.ds(sid * SLC, SLC)])

    start = wid * BPW
    pltpu.sync_copy(src_hbm.at[pl.ds(start, BPW)], idx_s)
    pltpu.sync_copy(dst_hbm.at[pl.ds(start, BPW)], idx_d)

    plsc.subcore_barrier()

    def scat(j, _):
        pltpu.sync_copy(sh_nin.at[idx_d.at[j]], vals.at[j])
        pltpu.sync_copy(vals.at[j], sh_c.at[idx_s.at[j]], add=True)
        return 0

    lax.fori_loop(0, BPW, scat, 0)

    plsc.subcore_barrier()

    pltpu.sync_copy(sh_c.at[pl.ds(sid * SLC, SLC)],
                    out_hbm.at[cid, pl.ds(sid * SLC, SLC)])


@functools.cache
def _cscatter_kernel():
    return pl.kernel(
        _cscatter_body,
        out_type=jax.ShapeDtypeStruct((NC, NPAD), jnp.float32),
        mesh=_mesh(),
        scratch_types=[
            pltpu.VMEM((BPW, BLK), jnp.int32),
            pltpu.VMEM((BPW, BLK), jnp.int32),
            pltpu.VMEM((BPW, BLK), jnp.float32),
            pltpu.VMEM((SLC,), jnp.float32),
            pltpu.VMEM_SHARED((NPAD,), jnp.float32),
            pltpu.VMEM_SHARED((NPAD,), jnp.float32),
        ],
    )


def _norm_body(deg_ref, nout_ref, nin_ref):
    x = deg_ref[...]
    deg_s = x[0, 0] + x[1, 0]
    deg_d = x[0, 1] + x[1, 1]
    nout_ref[...] = lax.rsqrt(jnp.maximum(deg_s, 1.0))
    nin_ref[...] = lax.rsqrt(jnp.maximum(deg_d, 1.0))


_ROWS = NPAD // 128


def _norms(deg_part):
    return pl.pallas_call(
        _norm_body,
        out_shape=[jax.ShapeDtypeStruct((_ROWS, 128), jnp.float32)] * 2,
    )(deg_part)


def _final_body(c0_ref, c1_ref, no_ref, f_ref, w2_ref, b2_ref, out_ref):
    j = pl.program_id(0)
    w = (c0_ref[...] + c1_ref[...]) * no_ref[...] * jnp.float32(1.0 / N)
    part = jnp.dot(w[0], f_ref[...], preferred_element_type=jnp.float32)

    @pl.when(j == 0)
    def _():
        out_ref[...] = jnp.zeros_like(out_ref)

    out_ref[...] += part

    @pl.when(j == _ROWS - 1)
    def _():
        out_ref[...] = (
            jnp.dot(out_ref[...], w2_ref[...], preferred_element_type=jnp.float32)
            + b2_ref[...]
        )


def _final(c0, c1, norm_out, f_pad, W2, b2):
    return pl.pallas_call(
        _final_body,
        grid=(_ROWS,),
        in_specs=[
            pl.BlockSpec((1, 1, 128), lambda j: (j, 0, 0)),
            pl.BlockSpec((1, 1, 128), lambda j: (j, 0, 0)),
            pl.BlockSpec((1, 1, 128), lambda j: (j, 0, 0)),
            pl.BlockSpec((128, 128), lambda j: (j, 0)),
            pl.BlockSpec((128, 128), lambda j: (0, 0)),
            pl.BlockSpec((1, 128), lambda j: (0, 0)),
        ],
        out_specs=pl.BlockSpec((1, 128), lambda j: (0, 0)),
        out_shape=jax.ShapeDtypeStruct((1, D), jnp.float32),
    )(c0, c1, norm_out, f_pad, W2, b2)


def kernel(feature, edge_index, W1, b1, W2, b2):
    pad = jnp.full((2, EPAD - E), SENT, jnp.int32)
    ei = jnp.concatenate([edge_index, pad], axis=1)
    src2 = ei[0].reshape(NBLKS, BLK)
    dst2 = ei[1].reshape(NBLKS, BLK)

    deg_part = _hist_kernel()(src2, dst2)
    norm_out, norm_in = _norms(deg_part.reshape(NC, 2, _ROWS, 128))
    c_part = _cscatter_kernel()(src2, dst2, norm_in.reshape(NPAD))

    f_pad = jnp.pad(feature, ((0, NPAD - N), (0, 0)))
    out = _final(
        c_part[0].reshape(_ROWS, 1, 128),
        c_part[1].reshape(_ROWS, 1, 128),
        norm_out.reshape(_ROWS, 1, 128),
        f_pad,
        W2,
        b2.reshape(1, D),
    )
    return out
